# baseline (speedup 1.0000x reference)
import functools

import jax
import jax.numpy as jnp
from jax import lax
from jax.experimental import pallas as pl
from jax.experimental.pallas import tpu as pltpu
from jax.experimental.pallas import tpu_sc as plsc

N_NODES = 10000
D = 128
BN_EPS = 1e-5

NC = 2
NS = 16
NW = NC * NS
CB = 128

N_PAD = 10240
ROWS_PER_TILE = N_PAD // NS
GARBAGE_ROW = N_NODES + 8


def _sc_aggregate(x, srcs, dsts, zeros):
    n_chunks = srcs.shape[1]
    mesh = plsc.VectorSubcoreMesh(core_axis_name="c", subcore_axis_name="s")

    @functools.partial(
        pl.kernel,
        mesh=mesh,
        out_type=jax.ShapeDtypeStruct((NC, N_PAD, D), jnp.float32),
        scratch_types=[
            pltpu.VMEM((n_chunks, CB), jnp.int32),
            pltpu.VMEM((n_chunks, CB), jnp.int32),
            pltpu.VMEM((CB, D), jnp.float32),
            pltpu.VMEM_SHARED((N_PAD, D), jnp.float32),
            pltpu.SemaphoreType.DMA,
        ],
    )
    def body(x_hbm, srcs_hbm, dsts_hbm, zeros_hbm, out_hbm,
             src_v, dst_v, rows0, agg_sh, sem0):
        c = lax.axis_index("c")
        s = lax.axis_index("s")
        w = c * NS + s
        pltpu.sync_copy(zeros_hbm, agg_sh.at[pl.ds(s * ROWS_PER_TILE,
                                                   ROWS_PER_TILE)])
        pltpu.sync_copy(srcs_hbm.at[w], src_v)
        pltpu.sync_copy(dsts_hbm.at[w], dst_v)
        plsc.subcore_barrier()

        def chunk(j, carry):
            pltpu.async_copy(x_hbm.at[src_v.at[j]], rows0, sem0).wait()
            pltpu.sync_copy(rows0, agg_sh.at[dst_v.at[j]], add=True)
            return carry

        lax.fori_loop(0, n_chunks, chunk, 0)
        plsc.subcore_barrier()
        pltpu.sync_copy(agg_sh.at[pl.ds(s * ROWS_PER_TILE, ROWS_PER_TILE)],
                        out_hbm.at[c, pl.ds(s * ROWS_PER_TILE, ROWS_PER_TILE)])

    return body(x, srcs, dsts, zeros)


BLK = 1000
N_BLKS = N_NODES // BLK


def _mlp_body(p_ref, x_ref, eps_ref, w1_ref, b1_ref, gamma_ref, beta_ref,
              w2_ref, b2_ref, y_ref, h_scr, stats_ref):
    p = pl.program_id(0)
    i = pl.program_id(1)

    @pl.when(p == 0)
    def _():
        out = p_ref[0] + p_ref[1] + (1.0 + eps_ref[0]) * x_ref[...]
        h = lax.dot_general(out, w1_ref[...], (((1,), (1,)), ((), ())),
                            preferred_element_type=jnp.float32) + b1_ref[...]
        h_scr[pl.ds(i * BLK, BLK), :] = h

        @pl.when(i == 0)
        def _():
            stats_ref[...] = jnp.zeros_like(stats_ref)

        stats_ref[0:1, :] += jnp.sum(h, axis=0, keepdims=True)
        stats_ref[1:2, :] += jnp.sum(h * h, axis=0, keepdims=True)
        y_ref[...] = jnp.zeros_like(y_ref)

    @pl.when(p == 1)
    def _():
        inv_n = 1.0 / N_NODES
        mean = stats_ref[0:1, :] * inv_n
        var = stats_ref[1:2, :] * inv_n - mean * mean
        inv = lax.rsqrt(var + BN_EPS)
        h = h_scr[pl.ds(i * BLK, BLK), :]
        hn = (h - mean) * (inv * gamma_ref[...]) + beta_ref[...]
        hr = jnp.maximum(hn, 0.0)
        y_ref[...] = lax.dot_general(hr, w2_ref[...],
                                     (((1,), (1,)), ((), ())),
                                     preferred_element_type=jnp.float32
                                     ) + b2_ref[...]


def kernel(x, edge_index, eps, W1, b1, gamma, beta, W2, b2):
    src = edge_index[0].astype(jnp.int32)
    dst = edge_index[1].astype(jnp.int32)
    e = src.shape[0]
    per_w = -(-e // (NW * CB)) * CB
    e_pad = per_w * NW
    pad_src = jnp.arange(e_pad - e, dtype=jnp.int32) % N_NODES
    srcs = jnp.concatenate([src, pad_src]).reshape(NW, per_w // CB, CB)
    pad_dst = N_NODES + jnp.arange(e_pad - e, dtype=jnp.int32) % (N_PAD - N_NODES)
    dsts = jnp.concatenate([dst, pad_dst]).reshape(NW, per_w // CB, CB)
    zeros = jnp.zeros((ROWS_PER_TILE, D), jnp.float32)

    partials = _sc_aggregate(x, srcs, dsts, zeros)

    y = pl.pallas_call(
        _mlp_body,
        grid=(2, N_BLKS),
        in_specs=[
            pl.BlockSpec((NC, BLK, D), lambda p, i: (0, i, 0)),
            pl.BlockSpec((BLK, D), lambda p, i: (i, 0)),
            pl.BlockSpec(memory_space=pltpu.MemorySpace.SMEM),
            pl.BlockSpec((D, D), lambda p, i: (0, 0)),
            pl.BlockSpec((1, D), lambda p, i: (0, 0)),
            pl.BlockSpec((1, D), lambda p, i: (0, 0)),
            pl.BlockSpec((1, D), lambda p, i: (0, 0)),
            pl.BlockSpec((D, D), lambda p, i: (0, 0)),
            pl.BlockSpec((1, D), lambda p, i: (0, 0)),
        ],
        out_specs=pl.BlockSpec((BLK, D), lambda p, i: (i, 0)),
        out_shape=jax.ShapeDtypeStruct((N_NODES, D), jnp.float32),
        scratch_shapes=[
            pltpu.VMEM((N_NODES, D), jnp.float32),
            pltpu.VMEM((8, D), jnp.float32),
        ],
    )(partials, x, eps, W1, b1.reshape(1, D), gamma.reshape(1, D),
      beta.reshape(1, D), W2, b2.reshape(1, D))

    return y


# speedup vs baseline: 1.0073x; 1.0073x over previous
import functools

import jax
import jax.numpy as jnp
from jax import lax
from jax.experimental import pallas as pl
from jax.experimental.pallas import tpu as pltpu
from jax.experimental.pallas import tpu_sc as plsc

N_NODES = 10000
D = 128
BN_EPS = 1e-5

NC = 2
NS = 16
NW = NC * NS
CB = 128

N_PAD = 10240
ROWS_PER_TILE = N_PAD // NS
GARBAGE_ROW = N_NODES + 8


def _sc_aggregate(x, srcs, dsts, zeros):
    n_chunks = srcs.shape[1]
    mesh = plsc.VectorSubcoreMesh(core_axis_name="c", subcore_axis_name="s")

    @functools.partial(
        pl.kernel,
        mesh=mesh,
        out_type=jax.ShapeDtypeStruct((NC, N_PAD, D), jnp.float32),
        scratch_types=[
            pltpu.VMEM((n_chunks, CB), jnp.int32),
            pltpu.VMEM((n_chunks, CB), jnp.int32),
            pltpu.VMEM((CB, D), jnp.float32),
            pltpu.VMEM_SHARED((N_PAD, D), jnp.float32),
            pltpu.SemaphoreType.DMA,
        ],
    )
    def body(x_hbm, srcs_hbm, dsts_hbm, zeros_hbm, out_hbm,
             src_v, dst_v, rows0, agg_sh, sem0):
        c = lax.axis_index("c")
        s = lax.axis_index("s")
        w = c * NS + s
        pltpu.sync_copy(zeros_hbm, agg_sh.at[pl.ds(s * ROWS_PER_TILE,
                                                   ROWS_PER_TILE)])
        pltpu.sync_copy(srcs_hbm.at[w], src_v)
        pltpu.sync_copy(dsts_hbm.at[w], dst_v)
        plsc.subcore_barrier()

        def chunk(j, carry):
            pltpu.async_copy(x_hbm.at[src_v.at[j]], rows0, sem0).wait()
            pltpu.sync_copy(rows0, agg_sh.at[dst_v.at[j]], add=True)
            return carry

        lax.fori_loop(0, n_chunks, chunk, 0)
        plsc.subcore_barrier()
        pltpu.sync_copy(agg_sh.at[pl.ds(s * ROWS_PER_TILE, ROWS_PER_TILE)],
                        out_hbm.at[c, pl.ds(s * ROWS_PER_TILE, ROWS_PER_TILE)])

    return body(x, srcs, dsts, zeros)


BLK = 1000
N_BLKS = N_NODES // BLK


def _mlp1_body(p_ref, x_ref, eps_ref, w1_ref, b1_ref, h_ref, stats_ref):
    i = pl.program_id(0)
    out = p_ref[0] + p_ref[1] + (1.0 + eps_ref[0]) * x_ref[...]
    h = lax.dot_general(out, w1_ref[...], (((1,), (1,)), ((), ())),
                        preferred_element_type=jnp.float32) + b1_ref[...]
    h_ref[...] = h

    @pl.when(i == 0)
    def _():
        stats_ref[...] = jnp.zeros_like(stats_ref)

    stats_ref[0:1, :] += jnp.sum(h, axis=0, keepdims=True)
    stats_ref[1:2, :] += jnp.sum(h * h, axis=0, keepdims=True)


def _mlp2_body(h_ref, stats_ref, gamma_ref, beta_ref, w2_ref, b2_ref, y_ref):
    inv_n = 1.0 / N_NODES
    mean = stats_ref[0:1, :] * inv_n
    var = stats_ref[1:2, :] * inv_n - mean * mean
    inv = lax.rsqrt(var + BN_EPS)
    hn = (h_ref[...] - mean) * (inv * gamma_ref[...]) + beta_ref[...]
    hr = jnp.maximum(hn, 0.0)
    y_ref[...] = lax.dot_general(hr, w2_ref[...], (((1,), (1,)), ((), ())),
                                 preferred_element_type=jnp.float32) + b2_ref[...]


def kernel(x, edge_index, eps, W1, b1, gamma, beta, W2, b2):
    src = edge_index[0].astype(jnp.int32)
    dst = edge_index[1].astype(jnp.int32)
    e = src.shape[0]
    per_w = -(-e // (NW * CB)) * CB
    e_pad = per_w * NW
    pad_src = jnp.arange(e_pad - e, dtype=jnp.int32) % N_NODES
    srcs = jnp.concatenate([src, pad_src]).reshape(NW, per_w // CB, CB)
    pad_dst = N_NODES + jnp.arange(e_pad - e, dtype=jnp.int32) % (N_PAD - N_NODES)
    dsts = jnp.concatenate([dst, pad_dst]).reshape(NW, per_w // CB, CB)
    zeros = jnp.zeros((ROWS_PER_TILE, D), jnp.float32)

    partials = _sc_aggregate(x, srcs, dsts, zeros)

    h, stats = pl.pallas_call(
        _mlp1_body,
        grid=(N_BLKS,),
        in_specs=[
            pl.BlockSpec((NC, BLK, D), lambda i: (0, i, 0)),
            pl.BlockSpec((BLK, D), lambda i: (i, 0)),
            pl.BlockSpec(memory_space=pltpu.MemorySpace.SMEM),
            pl.BlockSpec((D, D), lambda i: (0, 0)),
            pl.BlockSpec((1, D), lambda i: (0, 0)),
        ],
        out_specs=[
            pl.BlockSpec((BLK, D), lambda i: (i, 0)),
            pl.BlockSpec((8, D), lambda i: (0, 0)),
        ],
        out_shape=[
            jax.ShapeDtypeStruct((N_NODES, D), jnp.float32),
            jax.ShapeDtypeStruct((8, D), jnp.float32),
        ],
    )(partials, x, eps, W1, b1.reshape(1, D))

    y = pl.pallas_call(
        _mlp2_body,
        grid=(N_BLKS,),
        in_specs=[
            pl.BlockSpec((BLK, D), lambda i: (i, 0)),
            pl.BlockSpec((8, D), lambda i: (0, 0)),
            pl.BlockSpec((1, D), lambda i: (0, 0)),
            pl.BlockSpec((1, D), lambda i: (0, 0)),
            pl.BlockSpec((D, D), lambda i: (0, 0)),
            pl.BlockSpec((1, D), lambda i: (0, 0)),
        ],
        out_specs=pl.BlockSpec((BLK, D), lambda i: (i, 0)),
        out_shape=jax.ShapeDtypeStruct((N_NODES, D), jnp.float32),
    )(h, stats, gamma.reshape(1, D), beta.reshape(1, D), W2,
      b2.reshape(1, D))

    return y


# speedup vs baseline: 1.0106x; 1.0034x over previous
import functools

import jax
import jax.numpy as jnp
from jax import lax
from jax.experimental import pallas as pl
from jax.experimental.pallas import tpu as pltpu
from jax.experimental.pallas import tpu_sc as plsc

N_NODES = 10000
D = 128
BN_EPS = 1e-5

NC = 2
NS = 16
NW = NC * NS
CB = 128

N_PAD = 10240
ROWS_PER_TILE = N_PAD // NS


def _sc_aggregate(x, srcs, dsts, zeros):
    n_chunks = srcs.shape[1]
    mesh = plsc.VectorSubcoreMesh(core_axis_name="c", subcore_axis_name="s")

    @functools.partial(
        pl.kernel,
        mesh=mesh,
        out_type=jax.ShapeDtypeStruct((NC, N_PAD, D), jnp.float32),
        scratch_types=[
            pltpu.VMEM((n_chunks, CB), jnp.int32),
            pltpu.VMEM((n_chunks, CB), jnp.int32),
            pltpu.VMEM((CB, D), jnp.float32),
            pltpu.VMEM_SHARED((N_PAD, D), jnp.float32),
            pltpu.SemaphoreType.DMA,
        ],
    )
    def body(x_hbm, srcs_hbm, dsts_hbm, zeros_hbm, out_hbm,
             src_v, dst_v, rows0, agg_sh, sem0):
        c = lax.axis_index("c")
        s = lax.axis_index("s")
        w = c * NS + s
        pltpu.sync_copy(zeros_hbm, agg_sh.at[pl.ds(s * ROWS_PER_TILE,
                                                   ROWS_PER_TILE)])
        pltpu.sync_copy(srcs_hbm.at[w], src_v)
        pltpu.sync_copy(dsts_hbm.at[w], dst_v)
        plsc.subcore_barrier()

        def chunk(j, carry):
            pltpu.async_copy(x_hbm.at[src_v.at[j]], rows0, sem0).wait()
            pltpu.sync_copy(rows0, agg_sh.at[dst_v.at[j]], add=True)
            return carry

        lax.fori_loop(0, n_chunks, chunk, 0)
        plsc.subcore_barrier()
        pltpu.sync_copy(agg_sh.at[pl.ds(s * ROWS_PER_TILE, ROWS_PER_TILE)],
                        out_hbm.at[c, pl.ds(s * ROWS_PER_TILE, ROWS_PER_TILE)])

    return body(x, srcs, dsts, zeros)


BLK = 1000
N_BLKS = N_NODES // BLK


def _mlp1_body(p_ref, x_ref, eps_ref, w1_ref, b1_ref, h_ref, stats_ref):
    i = pl.program_id(0)
    out = p_ref[0] + p_ref[1] + (1.0 + eps_ref[0]) * x_ref[...]
    h = lax.dot_general(out, w1_ref[...], (((1,), (1,)), ((), ())),
                        preferred_element_type=jnp.float32) + b1_ref[...]
    h_ref[...] = h

    @pl.when(i == 0)
    def _():
        stats_ref[...] = jnp.zeros_like(stats_ref)

    stats_ref[0:1, :] += jnp.sum(h, axis=0, keepdims=True)
    stats_ref[1:2, :] += jnp.sum(h * h, axis=0, keepdims=True)


def _mlp2_body(h_ref, stats_ref, gamma_ref, beta_ref, w2_ref, b2_ref, y_ref):
    inv_n = 1.0 / N_NODES
    mean = stats_ref[0:1, :] * inv_n
    var = stats_ref[1:2, :] * inv_n - mean * mean
    inv = lax.rsqrt(var + BN_EPS)
    hn = (h_ref[...] - mean) * (inv * gamma_ref[...]) + beta_ref[...]
    hr = jnp.maximum(hn, 0.0)
    y_ref[...] = lax.dot_general(hr, w2_ref[...], (((1,), (1,)), ((), ())),
                                 preferred_element_type=jnp.float32) + b2_ref[...]


def kernel(x, edge_index, eps, W1, b1, gamma, beta, W2, b2):
    src = edge_index[0].astype(jnp.int32)
    dst = edge_index[1].astype(jnp.int32)
    e = src.shape[0]
    per_w = -(-e // (NW * CB)) * CB
    e_pad = per_w * NW
    pad_src = jnp.arange(e_pad - e, dtype=jnp.int32) % N_NODES
    srcs = jnp.concatenate([src, pad_src]).reshape(NW, per_w // CB, CB)
    pad_dst = N_NODES + jnp.arange(e_pad - e, dtype=jnp.int32) % (N_PAD - N_NODES)
    dsts = jnp.concatenate([dst, pad_dst]).reshape(NW, per_w // CB, CB)
    zeros = jnp.zeros((ROWS_PER_TILE, D), jnp.float32)

    partials = _sc_aggregate(x, srcs, dsts, zeros)

    h, stats = pl.pallas_call(
        _mlp1_body,
        grid=(N_BLKS,),
        in_specs=[
            pl.BlockSpec((NC, BLK, D), lambda i: (0, i, 0)),
            pl.BlockSpec((BLK, D), lambda i: (i, 0)),
            pl.BlockSpec(memory_space=pltpu.MemorySpace.SMEM),
            pl.BlockSpec((D, D), lambda i: (0, 0)),
            pl.BlockSpec((1, D), lambda i: (0, 0)),
        ],
        out_specs=[
            pl.BlockSpec((BLK, D), lambda i: (i, 0)),
            pl.BlockSpec((8, D), lambda i: (0, 0)),
        ],
        out_shape=[
            jax.ShapeDtypeStruct((N_NODES, D), jnp.float32),
            jax.ShapeDtypeStruct((8, D), jnp.float32),
        ],
    )(partials, x, eps, W1, b1.reshape(1, D))

    y = pl.pallas_call(
        _mlp2_body,
        grid=(N_BLKS,),
        in_specs=[
            pl.BlockSpec((BLK, D), lambda i: (i, 0)),
            pl.BlockSpec((8, D), lambda i: (0, 0)),
            pl.BlockSpec((1, D), lambda i: (0, 0)),
            pl.BlockSpec((1, D), lambda i: (0, 0)),
            pl.BlockSpec((D, D), lambda i: (0, 0)),
            pl.BlockSpec((1, D), lambda i: (0, 0)),
        ],
        out_specs=pl.BlockSpec((BLK, D), lambda i: (i, 0)),
        out_shape=jax.ShapeDtypeStruct((N_NODES, D), jnp.float32),
    )(h, stats, gamma.reshape(1, D), beta.reshape(1, D), W2,
      b2.reshape(1, D))

    return y
